# Initial kernel scaffold; baseline (speedup 1.0000x reference)
import functools

import jax
import jax.numpy as jnp
from jax import lax
from jax.experimental import pallas as pl
from jax.experimental.pallas import tpu as pltpu
from jax.experimental.pallas import tpu_sc as plsc

ALPHA = 0.1
K_STEPS = 2

NC = 2
NS = 16
CHUNK = 80
DEGW = 16


def _deg_body(src_hbm, dst_hbm, ones_hbm, zeros_hbm,
              outA, outB, inA, inB,
              idx_v, ones_v, outdeg_sh, indeg_sh, sem):
    cid = lax.axis_index("c")
    sid = lax.axis_index("s")
    E = src_hbm.shape[0]
    N = outdeg_sh.shape[0]
    ept = E // (NC * NS)
    nchunks = ept // CHUNK
    rows = N // NS
    row0 = sid * rows

    pltpu.sync_copy(ones_hbm, ones_v)
    pltpu.sync_copy(zeros_hbm.at[pl.ds(0, rows)], outdeg_sh.at[pl.ds(row0, rows)])
    pltpu.sync_copy(zeros_hbm.at[pl.ds(0, rows)], indeg_sh.at[pl.ds(row0, rows)])
    plsc.subcore_barrier()

    ebase = cid * (E // NC) + sid * ept

    def body(c, carry):
        base = ebase + c * CHUNK
        pltpu.sync_copy(src_hbm.at[pl.ds(base, CHUNK)], idx_v)
        pltpu.sync_copy(ones_v, outdeg_sh.at[idx_v], add=True)
        pltpu.sync_copy(dst_hbm.at[pl.ds(base, CHUNK)], idx_v)
        pltpu.sync_copy(ones_v, indeg_sh.at[idx_v], add=True)
        return carry

    lax.fori_loop(0, nchunks, body, 0)
    plsc.subcore_barrier()

    sl = pl.ds(row0, rows)

    @pl.when(cid == 0)
    def _():
        pltpu.sync_copy(outdeg_sh.at[sl], outA.at[sl])
        pltpu.sync_copy(indeg_sh.at[sl], inA.at[sl])

    @pl.when(cid == 1)
    def _():
        pltpu.sync_copy(outdeg_sh.at[sl], outB.at[sl])
        pltpu.sync_copy(indeg_sh.at[sl], inB.at[sl])


def _edge_body(hs_hbm, src_hbm, dst_hbm, zeros_hbm,
               aggA, aggB,
               sidx_v, didx_v, stage_v, agg_sh, sem):
    cid = lax.axis_index("c")
    sid = lax.axis_index("s")
    E = src_hbm.shape[0]
    N, D = agg_sh.shape
    ept = E // (NC * NS)
    nchunks = ept // CHUNK
    rows = N // NS
    row0 = sid * rows

    pltpu.sync_copy(zeros_hbm.at[pl.ds(0, rows)], agg_sh.at[pl.ds(row0, rows)])
    plsc.subcore_barrier()

    ebase = cid * (E // NC) + sid * ept

    def body(c, carry):
        base = ebase + c * CHUNK
        pltpu.sync_copy(src_hbm.at[pl.ds(base, CHUNK)], sidx_v)
        pltpu.async_copy(hs_hbm.at[sidx_v], stage_v, sem).wait()
        pltpu.sync_copy(dst_hbm.at[pl.ds(base, CHUNK)], didx_v)
        pltpu.sync_copy(stage_v, agg_sh.at[didx_v], add=True)
        return carry

    lax.fori_loop(0, nchunks, body, 0)
    plsc.subcore_barrier()

    sl = pl.ds(row0, rows)

    @pl.when(cid == 0)
    def _():
        pltpu.sync_copy(agg_sh.at[sl], aggA.at[sl])

    @pl.when(cid == 1)
    def _():
        pltpu.sync_copy(agg_sh.at[sl], aggB.at[sl])


def _linear_body(x_ref, w_ref, b_ref, oA_ref, oB_ref, iA_ref, iB_ref,
                 h0_ref, h0s_ref, ns_ref, nd_ref):
    h0 = jax.lax.dot_general(
        x_ref[...], w_ref[...], (((1,), (1,)), ((), ())),
        preferred_element_type=jnp.float32) + b_ref[...]
    ns = jax.lax.rsqrt(jnp.clip(oA_ref[...] + oB_ref[...], 1.0, None))
    nd = jax.lax.rsqrt(jnp.clip(iA_ref[...] + iB_ref[...], 1.0, None))
    h0_ref[...] = h0
    h0s_ref[...] = h0 * ns[:, 0:1]
    ns_ref[...] = ns
    nd_ref[...] = nd


def _blend_body(scale_src, aggA_ref, aggB_ref, h0_ref, ns_ref, nd_ref, out_ref):
    h = (1.0 - ALPHA) * nd_ref[:, 0:1] * (aggA_ref[...] + aggB_ref[...]) \
        + ALPHA * h0_ref[...]
    if scale_src:
        h = h * ns_ref[:, 0:1]
    out_ref[...] = h


def kernel(x, edge_index, W, b):
    N, D = x.shape
    E = edge_index.shape[1]
    src = edge_index[0]
    dst = edge_index[1]

    f32 = jnp.float32
    mesh = plsc.VectorSubcoreMesh(core_axis_name="c", subcore_axis_name="s")

    rows = N // NS
    ones_rows = jnp.zeros((CHUNK, DEGW), f32).at[:, 0].set(1.0)
    zeros_deg = jnp.zeros((rows, DEGW), f32)
    zeros_agg = jnp.zeros((rows, D), f32)

    deg_kernel = pl.kernel(
        _deg_body,
        out_type=[jax.ShapeDtypeStruct((N, DEGW), f32)] * 4,
        mesh=mesh,
        scratch_types=[
            pltpu.VMEM((CHUNK,), jnp.int32),
            pltpu.VMEM((CHUNK, DEGW), f32),
            pltpu.VMEM_SHARED((N, DEGW), f32),
            pltpu.VMEM_SHARED((N, DEGW), f32),
            pltpu.SemaphoreType.DMA,
        ],
    )
    outA, outB, inA, inB = deg_kernel(src, dst, ones_rows, zeros_deg)

    grid = 10
    blk = N // grid
    linear = pl.pallas_call(
        _linear_body,
        grid=(grid,),
        in_specs=[
            pl.BlockSpec((blk, D), lambda i: (i, 0)),
            pl.BlockSpec((D, D), lambda i: (0, 0)),
            pl.BlockSpec((1, D), lambda i: (0, 0)),
            pl.BlockSpec((blk, DEGW), lambda i: (i, 0)),
            pl.BlockSpec((blk, DEGW), lambda i: (i, 0)),
            pl.BlockSpec((blk, DEGW), lambda i: (i, 0)),
            pl.BlockSpec((blk, DEGW), lambda i: (i, 0)),
        ],
        out_specs=[
            pl.BlockSpec((blk, D), lambda i: (i, 0)),
            pl.BlockSpec((blk, D), lambda i: (i, 0)),
            pl.BlockSpec((blk, DEGW), lambda i: (i, 0)),
            pl.BlockSpec((blk, DEGW), lambda i: (i, 0)),
        ],
        out_shape=[
            jax.ShapeDtypeStruct((N, D), f32),
            jax.ShapeDtypeStruct((N, D), f32),
            jax.ShapeDtypeStruct((N, DEGW), f32),
            jax.ShapeDtypeStruct((N, DEGW), f32),
        ],
    )
    h0, h0s, ns, nd = linear(x, W, b.reshape(1, D), outA, outB, inA, inB)

    edge_kernel = pl.kernel(
        _edge_body,
        out_type=[jax.ShapeDtypeStruct((N, D), f32)] * 2,
        mesh=mesh,
        scratch_types=[
            pltpu.VMEM((CHUNK,), jnp.int32),
            pltpu.VMEM((CHUNK,), jnp.int32),
            pltpu.VMEM((CHUNK, D), f32),
            pltpu.VMEM_SHARED((N, D), f32),
            pltpu.SemaphoreType.DMA,
        ],
    )

    def blend(scale_src, aggA, aggB):
        return pl.pallas_call(
            functools.partial(_blend_body, scale_src),
            grid=(grid,),
            in_specs=[
                pl.BlockSpec((blk, D), lambda i: (i, 0)),
                pl.BlockSpec((blk, D), lambda i: (i, 0)),
                pl.BlockSpec((blk, D), lambda i: (i, 0)),
                pl.BlockSpec((blk, DEGW), lambda i: (i, 0)),
                pl.BlockSpec((blk, DEGW), lambda i: (i, 0)),
            ],
            out_specs=pl.BlockSpec((blk, D), lambda i: (i, 0)),
            out_shape=jax.ShapeDtypeStruct((N, D), f32),
        )(aggA, aggB, h0, ns, nd)

    h = h0s
    for step in range(K_STEPS):
        aggA, aggB = edge_kernel(h, src, dst, zeros_agg)
        h = blend(step < K_STEPS - 1, aggA, aggB)
    return h


# speedup vs baseline: 3.1538x; 3.1538x over previous
import functools

import jax
import jax.numpy as jnp
from jax import lax
from jax.experimental import pallas as pl
from jax.experimental.pallas import tpu as pltpu
from jax.experimental.pallas import tpu_sc as plsc

ALPHA = 0.1
K_STEPS = 2

NC = 2
NS = 16
CHUNK = 128
DEGW = 16


def _deg_body(edges_hbm, ones_hbm, zeros_hbm, deg_hbm,
              idx_v, ones_v, tab_sh, sem):
    cid = lax.axis_index("c")
    sid = lax.axis_index("s")
    E = edges_hbm.shape[1]
    N = tab_sh.shape[0]
    ept = E // NS
    nchunks = ept // CHUNK
    rows = N // NS
    row0 = sid * rows

    pltpu.sync_copy(ones_hbm, ones_v)
    pltpu.sync_copy(zeros_hbm, tab_sh.at[pl.ds(row0, rows)])
    plsc.subcore_barrier()

    ebase = sid * ept

    def body(c, carry):
        base = ebase + c * CHUNK
        pltpu.sync_copy(edges_hbm.at[cid].at[pl.ds(base, CHUNK)], idx_v)
        pltpu.sync_copy(ones_v, tab_sh.at[idx_v], add=True)
        return carry

    lax.fori_loop(0, nchunks, body, 0)
    plsc.subcore_barrier()

    sl = pl.ds(row0, rows)
    pltpu.sync_copy(tab_sh.at[sl], deg_hbm.at[cid].at[sl])


def _edge_body(hs_hbm, src_hbm, dst_hbm, zeros_hbm,
               agg_hbm,
               sidx_v, didx_v, stage_v, agg_sh, sem):
    cid = lax.axis_index("c")
    sid = lax.axis_index("s")
    E = src_hbm.shape[0]
    N, D = agg_sh.shape
    ept = E // (NC * NS)
    nchunks = ept // CHUNK
    rows = N // NS
    row0 = sid * rows

    pltpu.sync_copy(zeros_hbm.at[pl.ds(0, rows)], agg_sh.at[pl.ds(row0, rows)])
    plsc.subcore_barrier()

    ebase = cid * (E // NC) + sid * ept

    def body(c, carry):
        base = ebase + c * CHUNK
        pltpu.sync_copy(src_hbm.at[pl.ds(base, CHUNK)], sidx_v)
        pltpu.async_copy(hs_hbm.at[sidx_v], stage_v, sem).wait()
        pltpu.sync_copy(dst_hbm.at[pl.ds(base, CHUNK)], didx_v)
        pltpu.sync_copy(stage_v, agg_sh.at[didx_v], add=True)
        return carry

    lax.fori_loop(0, nchunks, body, 0)
    plsc.subcore_barrier()

    sl = pl.ds(row0, rows)
    pltpu.sync_copy(agg_sh.at[sl], agg_hbm.at[cid].at[sl])


def _linear_body(x_ref, w_ref, b_ref, od_ref, id_ref,
                 h0_ref, h0s_ref, ns_ref, nd_ref):
    h0 = jax.lax.dot_general(
        x_ref[...], w_ref[...], (((1,), (1,)), ((), ())),
        preferred_element_type=jnp.float32) + b_ref[...]
    ns = jax.lax.rsqrt(jnp.clip(od_ref[:, 0:1], 1.0, None))
    nd = jax.lax.rsqrt(jnp.clip(id_ref[:, 0:1], 1.0, None))
    h0_ref[...] = h0
    h0s_ref[...] = h0 * ns
    ns_ref[...] = jnp.broadcast_to(ns, ns_ref.shape)
    nd_ref[...] = jnp.broadcast_to(nd, nd_ref.shape)


def _blend_body(scale_src, aggA_ref, aggB_ref, h0_ref, ns_ref, nd_ref, out_ref):
    h = (1.0 - ALPHA) * nd_ref[:, 0:1] * (aggA_ref[...] + aggB_ref[...]) \
        + ALPHA * h0_ref[...]
    if scale_src:
        h = h * ns_ref[:, 0:1]
    out_ref[...] = h


def kernel(x, edge_index, W, b):
    N0, D = x.shape
    E0 = edge_index.shape[1]

    f32 = jnp.float32
    mesh = plsc.VectorSubcoreMesh(core_axis_name="c", subcore_axis_name="s")

    quantum = NS * 8 * 10
    N = ((N0 + quantum - 1) // quantum) * quantum
    x = jnp.pad(x, ((0, N - N0), (0, 0)))

    equantum = NC * NS * CHUNK
    E = ((E0 + equantum - 1) // equantum) * equantum
    edges = jnp.pad(edge_index, ((0, 0), (0, E - E0)), constant_values=N - 1)
    src = edges[0]
    dst = edges[1]

    rows = N // NS
    ones_rows = jnp.zeros((CHUNK, D), f32).at[:, 0].set(1.0)
    zeros_agg = jnp.zeros((rows, D), f32)

    deg_kernel = pl.kernel(
        _deg_body,
        out_type=jax.ShapeDtypeStruct((NC, N, D), f32),
        mesh=mesh,
        scratch_types=[
            pltpu.VMEM((CHUNK,), jnp.int32),
            pltpu.VMEM((CHUNK, D), f32),
            pltpu.VMEM_SHARED((N, D), f32),
            pltpu.SemaphoreType.DMA,
        ],
    )
    deg2 = deg_kernel(edges, ones_rows, zeros_agg)
    outdeg, indeg = deg2[0], deg2[1]

    grid = 10
    blk = N // grid
    assert N % grid == 0
    linear = pl.pallas_call(
        _linear_body,
        grid=(grid,),
        in_specs=[
            pl.BlockSpec((blk, D), lambda i: (i, 0)),
            pl.BlockSpec((D, D), lambda i: (0, 0)),
            pl.BlockSpec((1, D), lambda i: (0, 0)),
            pl.BlockSpec((blk, D), lambda i: (i, 0)),
            pl.BlockSpec((blk, D), lambda i: (i, 0)),
        ],
        out_specs=[
            pl.BlockSpec((blk, D), lambda i: (i, 0)),
            pl.BlockSpec((blk, D), lambda i: (i, 0)),
            pl.BlockSpec((blk, DEGW), lambda i: (i, 0)),
            pl.BlockSpec((blk, DEGW), lambda i: (i, 0)),
        ],
        out_shape=[
            jax.ShapeDtypeStruct((N, D), f32),
            jax.ShapeDtypeStruct((N, D), f32),
            jax.ShapeDtypeStruct((N, DEGW), f32),
            jax.ShapeDtypeStruct((N, DEGW), f32),
        ],
    )
    h0, h0s, ns, nd = linear(x, W, b.reshape(1, D), outdeg, indeg)

    edge_kernel = pl.kernel(
        _edge_body,
        out_type=jax.ShapeDtypeStruct((NC, N, D), f32),
        mesh=mesh,
        scratch_types=[
            pltpu.VMEM((CHUNK,), jnp.int32),
            pltpu.VMEM((CHUNK,), jnp.int32),
            pltpu.VMEM((CHUNK, D), f32),
            pltpu.VMEM_SHARED((N, D), f32),
            pltpu.SemaphoreType.DMA,
        ],
    )

    def blend(scale_src, aggA, aggB):
        return pl.pallas_call(
            functools.partial(_blend_body, scale_src),
            grid=(grid,),
            in_specs=[
                pl.BlockSpec((blk, D), lambda i: (i, 0)),
                pl.BlockSpec((blk, D), lambda i: (i, 0)),
                pl.BlockSpec((blk, D), lambda i: (i, 0)),
                pl.BlockSpec((blk, DEGW), lambda i: (i, 0)),
                pl.BlockSpec((blk, DEGW), lambda i: (i, 0)),
            ],
            out_specs=pl.BlockSpec((blk, D), lambda i: (i, 0)),
            out_shape=jax.ShapeDtypeStruct((N, D), f32),
        )(aggA, aggB, h0, ns, nd)

    h = h0s
    for step in range(K_STEPS):
        agg2 = edge_kernel(h, src, dst, zeros_agg)
        h = blend(step < K_STEPS - 1, agg2[0], agg2[1])
    return h[:N0]
